# Initial kernel scaffold; baseline (speedup 1.0000x reference)
import jax
import jax.numpy as jnp
from jax.experimental import pallas as pl


def kernel(index, names_table):
    raise NotImplementedError("write your pallas kernel here")


# speedup vs baseline: 177.4498x; 177.4498x over previous
import functools

import jax
import jax.numpy as jnp
from jax import lax
from jax.experimental import pallas as pl
from jax.experimental.pallas import tpu as pltpu
from jax.experimental.pallas import tpu_sc as plsc

NUM_CORES = 2
NUM_SUBCORES = 16
LANES = 16
NW = NUM_CORES * NUM_SUBCORES

TOTAL = 16384 * 200
PER_WORKER = TOTAL // NW
CHUNK = 51200
N_CHUNKS = PER_WORKER // CHUNK
VOCAB_PAD = 1024


def _make_sc_gather(vocab):
    mesh = plsc.VectorSubcoreMesh(
        core_axis_name="c", subcore_axis_name="s", num_cores=NUM_CORES
    )

    @functools.partial(
        pl.kernel,
        mesh=mesh,
        out_type=jax.ShapeDtypeStruct((TOTAL,), jnp.float32),
        scratch_types=[
            pltpu.VMEM((VOCAB_PAD,), jnp.float32),
            pltpu.VMEM((CHUNK,), jnp.int32),
            pltpu.VMEM((CHUNK,), jnp.float32),
        ],
        compiler_params=pltpu.CompilerParams(needs_layout_passes=False),
    )
    def sc_gather(table_hbm, idx_hbm, out_hbm, tab_v, idx_v, val_v):
        wid = lax.axis_index("s") * NUM_CORES + lax.axis_index("c")
        base = wid * PER_WORKER
        pltpu.sync_copy(table_hbm, tab_v.at[pl.ds(0, vocab)])

        def do_chunk(c, _):
            off = base + c * CHUNK
            pltpu.sync_copy(idx_hbm.at[pl.ds(off, CHUNK)], idx_v)

            def gather_step(i, _):
                s = i * LANES
                iv = idx_v[pl.ds(s, LANES)]
                val_v[pl.ds(s, LANES)] = plsc.load_gather(tab_v, [iv])
                return 0

            lax.fori_loop(0, CHUNK // LANES, gather_step, 0)
            pltpu.sync_copy(val_v, out_hbm.at[pl.ds(off, CHUNK)])
            return 0

        lax.fori_loop(0, N_CHUNKS, do_chunk, 0)

    return sc_gather


_sc_gather_1000 = _make_sc_gather(1000)


def kernel(index, names_table):
    flat_idx = index.reshape(-1)
    out = _sc_gather_1000(names_table, flat_idx)
    return out.reshape(index.shape)


# speedup vs baseline: 223.7307x; 1.2608x over previous
import functools

import jax
import jax.numpy as jnp
from jax import lax
from jax.experimental import pallas as pl
from jax.experimental.pallas import tpu as pltpu
from jax.experimental.pallas import tpu_sc as plsc

NUM_CORES = 2
NUM_SUBCORES = 16
LANES = 16
NW = NUM_CORES * NUM_SUBCORES

TOTAL = 16384 * 200
PER_WORKER = TOTAL // NW
CHUNK = 51200
N_CHUNKS = PER_WORKER // CHUNK
VOCAB_PAD = 1024


def _make_sc_gather(vocab):
    mesh = plsc.VectorSubcoreMesh(
        core_axis_name="c", subcore_axis_name="s", num_cores=NUM_CORES
    )

    @functools.partial(
        pl.kernel,
        mesh=mesh,
        out_type=jax.ShapeDtypeStruct((TOTAL,), jnp.float32),
        scratch_types=[
            pltpu.VMEM((VOCAB_PAD,), jnp.float32),
            pltpu.VMEM((CHUNK,), jnp.int32),
            pltpu.VMEM((CHUNK,), jnp.float32),
        ],
        compiler_params=pltpu.CompilerParams(needs_layout_passes=False),
    )
    def sc_gather(table_hbm, idx_hbm, out_hbm, tab_v, idx_v, val_v):
        wid = lax.axis_index("s") * NUM_CORES + lax.axis_index("c")
        base = wid * PER_WORKER
        pltpu.sync_copy(table_hbm, tab_v.at[pl.ds(0, vocab)])

        def do_chunk(c, _):
            off = base + c * CHUNK
            pltpu.sync_copy(idx_hbm.at[pl.ds(off, CHUNK)], idx_v)

            def gather_step(i):
                s = i * LANES
                iv = idx_v[pl.ds(s, LANES)]
                val_v[pl.ds(s, LANES)] = plsc.load_gather(tab_v, [iv])

            plsc.parallel_loop(0, CHUNK // LANES, unroll=8)(gather_step)
            pltpu.sync_copy(val_v, out_hbm.at[pl.ds(off, CHUNK)])
            return 0

        lax.fori_loop(0, N_CHUNKS, do_chunk, 0)

    return sc_gather


_sc_gather_1000 = _make_sc_gather(1000)


def kernel(index, names_table):
    flat_idx = index.reshape(-1)
    out = _sc_gather_1000(names_table, flat_idx)
    return out.reshape(index.shape)


# speedup vs baseline: 371.6605x; 1.6612x over previous
import functools

import jax
import jax.numpy as jnp
from jax import lax
from jax.experimental import pallas as pl
from jax.experimental.pallas import tpu as pltpu
from jax.experimental.pallas import tpu_sc as plsc

NUM_CORES = 2
NUM_SUBCORES = 16
LANES = 16
NW = NUM_CORES * NUM_SUBCORES

ROWS = 16384
COLS = 200
ROWS_PER_WORKER = ROWS // NW
CHUNK_ROWS = 128
N_CHUNKS = ROWS_PER_WORKER // CHUNK_ROWS
VOCAB_PAD = 1024

_COL_OFFS = [k * LANES for k in range(COLS // LANES)] + [COLS - LANES]


def _make_sc_gather(vocab):
    mesh = plsc.VectorSubcoreMesh(
        core_axis_name="c", subcore_axis_name="s", num_cores=NUM_CORES
    )

    @functools.partial(
        pl.kernel,
        mesh=mesh,
        out_type=jax.ShapeDtypeStruct((ROWS, COLS), jnp.float32),
        scratch_types=[
            pltpu.VMEM((VOCAB_PAD,), jnp.float32),
            pltpu.VMEM((CHUNK_ROWS, COLS), jnp.int32),
            pltpu.VMEM((CHUNK_ROWS, COLS), jnp.float32),
        ],
        compiler_params=pltpu.CompilerParams(needs_layout_passes=False),
    )
    def sc_gather(table_hbm, idx_hbm, out_hbm, tab_v, idx_v, val_v):
        wid = lax.axis_index("s") * NUM_CORES + lax.axis_index("c")
        base = wid * ROWS_PER_WORKER
        pltpu.sync_copy(table_hbm, tab_v.at[pl.ds(0, vocab)])

        def do_chunk(c, _):
            r0 = base + c * CHUNK_ROWS
            pltpu.sync_copy(idx_hbm.at[pl.ds(r0, CHUNK_ROWS)], idx_v)

            def gather_row(r):
                for off in _COL_OFFS:
                    iv = idx_v[r, pl.ds(off, LANES)]
                    val_v[r, pl.ds(off, LANES)] = plsc.load_gather(
                        tab_v, [iv]
                    )

            plsc.parallel_loop(0, CHUNK_ROWS, unroll=2)(gather_row)
            pltpu.sync_copy(val_v, out_hbm.at[pl.ds(r0, CHUNK_ROWS)])
            return 0

        lax.fori_loop(0, N_CHUNKS, do_chunk, 0)

    return sc_gather


_sc_gather_1000 = _make_sc_gather(1000)


def kernel(index, names_table):
    return _sc_gather_1000(names_table, index)


# speedup vs baseline: 405.0199x; 1.0898x over previous
import functools

import jax
import jax.numpy as jnp
from jax import lax
from jax.experimental import pallas as pl
from jax.experimental.pallas import tpu as pltpu
from jax.experimental.pallas import tpu_sc as plsc

NUM_CORES = 2
NUM_SUBCORES = 16
LANES = 16
NW = NUM_CORES * NUM_SUBCORES

ROWS = 16384
COLS = 200
ROWS_PER_WORKER = ROWS // NW
CHUNK_ROWS = 64
N_CHUNKS = ROWS_PER_WORKER // CHUNK_ROWS
VOCAB_PAD = 1024

_COL_OFFS = [k * LANES for k in range(COLS // LANES)] + [COLS - LANES]


def _make_sc_gather(vocab):
    mesh = plsc.VectorSubcoreMesh(
        core_axis_name="c", subcore_axis_name="s", num_cores=NUM_CORES
    )

    @functools.partial(
        pl.kernel,
        mesh=mesh,
        out_type=jax.ShapeDtypeStruct((ROWS, COLS), jnp.float32),
        scratch_types=[
            pltpu.VMEM((VOCAB_PAD,), jnp.float32),
            pltpu.VMEM((2, CHUNK_ROWS, COLS), jnp.int32),
            pltpu.VMEM((2, CHUNK_ROWS, COLS), jnp.float32),
            pltpu.SemaphoreType.DMA,
            pltpu.SemaphoreType.DMA,
            pltpu.SemaphoreType.DMA,
            pltpu.SemaphoreType.DMA,
        ],
        compiler_params=pltpu.CompilerParams(needs_layout_passes=False),
    )
    def sc_gather(
        table_hbm, idx_hbm, out_hbm, tab_v, idx_v, val_v,
        sem_i0, sem_i1, sem_o0, sem_o1,
    ):
        sem_i = (sem_i0, sem_i1)
        sem_o = (sem_o0, sem_o1)
        wid = lax.axis_index("s") * NUM_CORES + lax.axis_index("c")
        base = wid * ROWS_PER_WORKER
        pltpu.sync_copy(table_hbm, tab_v.at[pl.ds(0, vocab)])

        def start_in(c):
            r0 = base + c * CHUNK_ROWS
            return pltpu.async_copy(
                idx_hbm.at[pl.ds(r0, CHUNK_ROWS)], idx_v.at[c % 2], sem_i[c % 2]
            )

        def start_out(c):
            r0 = base + c * CHUNK_ROWS
            return pltpu.async_copy(
                val_v.at[c % 2], out_hbm.at[pl.ds(r0, CHUNK_ROWS)], sem_o[c % 2]
            )

        in_dma = {0: start_in(0)}
        out_dma = {}
        for c in range(N_CHUNKS):
            b = c % 2
            if c + 1 < N_CHUNKS:
                in_dma[c + 1] = start_in(c + 1)
            in_dma[c].wait()
            if c >= 2:
                out_dma[c - 2].wait()
            idx_b = idx_v.at[b]
            val_b = val_v.at[b]

            def gather_row(r):
                for off in _COL_OFFS:
                    iv = idx_b[r, pl.ds(off, LANES)]
                    val_b[r, pl.ds(off, LANES)] = plsc.load_gather(
                        tab_v, [iv]
                    )

            plsc.parallel_loop(0, CHUNK_ROWS, unroll=2)(gather_row)
            out_dma[c] = start_out(c)
        out_dma[N_CHUNKS - 2].wait()
        out_dma[N_CHUNKS - 1].wait()

    return sc_gather


_sc_gather_1000 = _make_sc_gather(1000)


def kernel(index, names_table):
    return _sc_gather_1000(names_table, index)


# speedup vs baseline: 405.5553x; 1.0013x over previous
import functools

import jax
import jax.numpy as jnp
from jax import lax
from jax.experimental import pallas as pl
from jax.experimental.pallas import tpu as pltpu
from jax.experimental.pallas import tpu_sc as plsc

NUM_CORES = 2
NUM_SUBCORES = 16
LANES = 16
NW = NUM_CORES * NUM_SUBCORES

ROWS = 16384
COLS = 200
ROWS_PER_WORKER = ROWS // NW
CHUNK_ROWS = 64
N_CHUNKS = ROWS_PER_WORKER // CHUNK_ROWS
VOCAB_PAD = 1024

_COL_OFFS = [k * LANES for k in range(COLS // LANES)] + [COLS - LANES]


def _make_sc_gather(vocab):
    mesh = plsc.VectorSubcoreMesh(
        core_axis_name="c", subcore_axis_name="s", num_cores=NUM_CORES
    )

    @functools.partial(
        pl.kernel,
        mesh=mesh,
        out_type=jax.ShapeDtypeStruct((ROWS, COLS), jnp.float32),
        scratch_types=[
            pltpu.VMEM((VOCAB_PAD,), jnp.float32),
            pltpu.VMEM((2, CHUNK_ROWS, COLS), jnp.int32),
            pltpu.VMEM((2, CHUNK_ROWS, COLS), jnp.float32),
            pltpu.SemaphoreType.DMA,
            pltpu.SemaphoreType.DMA,
            pltpu.SemaphoreType.DMA,
            pltpu.SemaphoreType.DMA,
        ],
        compiler_params=pltpu.CompilerParams(
            needs_layout_passes=False, use_tc_tiling_on_sc=True
        ),
    )
    def sc_gather(
        table_hbm, idx_hbm, out_hbm, tab_v, idx_v, val_v,
        sem_i0, sem_i1, sem_o0, sem_o1,
    ):
        sem_i = (sem_i0, sem_i1)
        sem_o = (sem_o0, sem_o1)
        wid = lax.axis_index("s") * NUM_CORES + lax.axis_index("c")
        base = wid * ROWS_PER_WORKER
        pltpu.sync_copy(table_hbm, tab_v.at[pl.ds(0, vocab)])

        def start_in(c):
            r0 = base + c * CHUNK_ROWS
            return pltpu.async_copy(
                idx_hbm.at[pl.ds(r0, CHUNK_ROWS)], idx_v.at[c % 2], sem_i[c % 2]
            )

        def start_out(c):
            r0 = base + c * CHUNK_ROWS
            return pltpu.async_copy(
                val_v.at[c % 2], out_hbm.at[pl.ds(r0, CHUNK_ROWS)], sem_o[c % 2]
            )

        in_dma = {0: start_in(0)}
        out_dma = {}
        for c in range(N_CHUNKS):
            b = c % 2
            if c + 1 < N_CHUNKS:
                in_dma[c + 1] = start_in(c + 1)
            in_dma[c].wait()
            if c >= 2:
                out_dma[c - 2].wait()
            idx_b = idx_v.at[b]
            val_b = val_v.at[b]

            def gather_row(r):
                for off in _COL_OFFS:
                    iv = idx_b[r, pl.ds(off, LANES)]
                    val_b[r, pl.ds(off, LANES)] = plsc.load_gather(
                        tab_v, [iv]
                    )

            plsc.parallel_loop(0, CHUNK_ROWS, unroll=2)(gather_row)
            out_dma[c] = start_out(c)
        out_dma[N_CHUNKS - 2].wait()
        out_dma[N_CHUNKS - 1].wait()

    return sc_gather


_sc_gather_1000 = _make_sc_gather(1000)


def kernel(index, names_table):
    return _sc_gather_1000(names_table, index)


# speedup vs baseline: 745.4960x; 1.8382x over previous
import functools

import jax
import jax.numpy as jnp
from jax import lax
from jax.experimental import pallas as pl
from jax.experimental.pallas import tpu as pltpu
from jax.experimental.pallas import tpu_sc as plsc

NUM_CORES = 2
NUM_SUBCORES = 16
LANES = 16
NW = NUM_CORES * NUM_SUBCORES

TROWS = 200
TCOLS = 16384
COLS_PER_WORKER = TCOLS // NW
CHUNK_COLS = 128
N_CHUNKS = COLS_PER_WORKER // CHUNK_COLS
VOCAB_PAD = 1024


def _make_sc_gather(vocab):
    mesh = plsc.VectorSubcoreMesh(
        core_axis_name="c", subcore_axis_name="s", num_cores=NUM_CORES
    )

    @functools.partial(
        pl.kernel,
        mesh=mesh,
        out_type=jax.ShapeDtypeStruct((TROWS, TCOLS), jnp.float32),
        scratch_types=[
            pltpu.VMEM((VOCAB_PAD,), jnp.float32),
            pltpu.VMEM((2, TROWS, CHUNK_COLS), jnp.int32),
            pltpu.VMEM((2, TROWS, CHUNK_COLS), jnp.float32),
            pltpu.SemaphoreType.DMA,
            pltpu.SemaphoreType.DMA,
            pltpu.SemaphoreType.DMA,
            pltpu.SemaphoreType.DMA,
        ],
        compiler_params=pltpu.CompilerParams(
            needs_layout_passes=False, use_tc_tiling_on_sc=True
        ),
    )
    def sc_gather(
        table_hbm, idx_hbm, out_hbm, tab_v, idx_v, val_v,
        sem_i0, sem_i1, sem_o0, sem_o1,
    ):
        sem_i = (sem_i0, sem_i1)
        sem_o = (sem_o0, sem_o1)
        wid = lax.axis_index("s") * NUM_CORES + lax.axis_index("c")
        base = wid * COLS_PER_WORKER
        pltpu.sync_copy(table_hbm, tab_v.at[pl.ds(0, vocab)])

        def start_in(c):
            c0 = base + c * CHUNK_COLS
            return pltpu.async_copy(
                idx_hbm.at[:, pl.ds(c0, CHUNK_COLS)],
                idx_v.at[c % 2],
                sem_i[c % 2],
            )

        def start_out(c):
            c0 = base + c * CHUNK_COLS
            return pltpu.async_copy(
                val_v.at[c % 2],
                out_hbm.at[:, pl.ds(c0, CHUNK_COLS)],
                sem_o[c % 2],
            )

        in_dma = {0: start_in(0)}
        out_dma = {}
        for c in range(N_CHUNKS):
            b = c % 2
            if c + 1 < N_CHUNKS:
                in_dma[c + 1] = start_in(c + 1)
            in_dma[c].wait()
            if c >= 2:
                out_dma[c - 2].wait()
            idx_b = idx_v.at[b]
            val_b = val_v.at[b]

            def gather_row(r):
                for k in range(CHUNK_COLS // LANES):
                    iv = idx_b[r, pl.ds(k * LANES, LANES)]
                    val_b[r, pl.ds(k * LANES, LANES)] = plsc.load_gather(
                        tab_v, [iv]
                    )

            plsc.parallel_loop(0, TROWS, unroll=2)(gather_row)
            out_dma[c] = start_out(c)
        out_dma[N_CHUNKS - 2].wait()
        out_dma[N_CHUNKS - 1].wait()

    return sc_gather


_sc_gather_1000 = _make_sc_gather(1000)


def kernel(index, names_table):
    out_t = _sc_gather_1000(names_table, index.T)
    return out_t.T


# speedup vs baseline: 770.9421x; 1.0341x over previous
import functools

import jax
import jax.numpy as jnp
from jax import lax
from jax.experimental import pallas as pl
from jax.experimental.pallas import tpu as pltpu
from jax.experimental.pallas import tpu_sc as plsc

NUM_CORES = 2
NUM_SUBCORES = 16
LANES = 16
NW = NUM_CORES * NUM_SUBCORES

TROWS = 200
TCOLS = 16384
COLS_PER_WORKER = TCOLS // NW
CHUNK_COLS = 128
N_CHUNKS = COLS_PER_WORKER // CHUNK_COLS
VOCAB_PAD = 1024


def _make_sc_gather(vocab):
    mesh = plsc.VectorSubcoreMesh(
        core_axis_name="c", subcore_axis_name="s", num_cores=NUM_CORES
    )

    @functools.partial(
        pl.kernel,
        mesh=mesh,
        out_type=jax.ShapeDtypeStruct((TROWS, TCOLS), jnp.float32),
        scratch_types=[
            pltpu.VMEM((VOCAB_PAD,), jnp.float32),
            pltpu.VMEM((2, TROWS, CHUNK_COLS), jnp.int32),
            pltpu.VMEM((2, TROWS, CHUNK_COLS), jnp.float32),
            pltpu.SemaphoreType.DMA,
            pltpu.SemaphoreType.DMA,
            pltpu.SemaphoreType.DMA,
            pltpu.SemaphoreType.DMA,
        ],
        compiler_params=pltpu.CompilerParams(
            needs_layout_passes=False, use_tc_tiling_on_sc=True
        ),
    )
    def sc_gather(
        table_hbm, idx_hbm, out_hbm, tab_v, idx_v, val_v,
        sem_i0, sem_i1, sem_o0, sem_o1,
    ):
        sem_i = (sem_i0, sem_i1)
        sem_o = (sem_o0, sem_o1)
        wid = lax.axis_index("s") * NUM_CORES + lax.axis_index("c")
        base = wid * COLS_PER_WORKER
        def start_in(c):
            c0 = base + c * CHUNK_COLS
            return pltpu.async_copy(
                idx_hbm.at[:, pl.ds(c0, CHUNK_COLS)],
                idx_v.at[c % 2],
                sem_i[c % 2],
            )

        def start_out(c):
            c0 = base + c * CHUNK_COLS
            return pltpu.async_copy(
                val_v.at[c % 2],
                out_hbm.at[:, pl.ds(c0, CHUNK_COLS)],
                sem_o[c % 2],
            )

        in_dma = {0: start_in(0)}
        out_dma = {}
        pltpu.sync_copy(table_hbm, tab_v.at[pl.ds(0, vocab)])
        for c in range(N_CHUNKS):
            b = c % 2
            if c + 1 < N_CHUNKS:
                in_dma[c + 1] = start_in(c + 1)
            in_dma[c].wait()
            if c >= 2:
                out_dma[c - 2].wait()
            idx_b = idx_v.at[b]
            val_b = val_v.at[b]

            def gather_row(r):
                for k in range(CHUNK_COLS // LANES):
                    iv = idx_b[r, pl.ds(k * LANES, LANES)]
                    val_b[r, pl.ds(k * LANES, LANES)] = plsc.load_gather(
                        tab_v, [iv]
                    )

            plsc.parallel_loop(0, TROWS, unroll=4)(gather_row)
            out_dma[c] = start_out(c)
        out_dma[N_CHUNKS - 2].wait()
        out_dma[N_CHUNKS - 1].wait()

    return sc_gather


_sc_gather_1000 = _make_sc_gather(1000)


def kernel(index, names_table):
    out_t = _sc_gather_1000(names_table, index.T)
    return out_t.T
